# baseline (speedup 1.0000x reference)
import dataclasses
import functools

import jax
import jax.numpy as jnp
from jax import lax
from jax.experimental import pallas as pl
from jax.experimental.pallas import tpu as pltpu
from jax.experimental.pallas import tpu_sc as plsc

NC = 2
NS = 16
NW = NC * NS
L16 = 16

HID = 64
KV = HID // L16

DEMO_VOCAB = 128
MAX_POS = 512
SEG_VOCAB = 2

C = 128


def _rsqrt(x):
    i = lax.bitcast_convert_type(x, jnp.int32)
    i = jnp.int32(0x5F3759DF) - lax.shift_right_arithmetic(i, 1)
    y = lax.bitcast_convert_type(i, jnp.float32)
    for _ in range(3):
        y = y * (1.5 - 0.5 * x * y * y)
    return y


def _splat(v, j):
    return v.at[jnp.full((L16,), j, dtype=jnp.int32)].get(
        mode="promise_in_bounds")


@functools.partial(jax.jit, static_argnames=("n_tok",))
def _embed_ln(n_tok, wids2, aids2, bids2, cids2, sids2, pids2,
              wtab, dtab_f, ptab_f, stab_f, gamma, beta):
    tok_w = n_tok // NW
    nchunk = tok_w // C
    rows_w = nchunk
    mesh = plsc.VectorSubcoreMesh(core_axis_name="c", subcore_axis_name="s")
    cp = pltpu.CompilerParams()
    if "needs_layout_passes" in pltpu.CompilerParams.__dataclass_fields__:
        cp = dataclasses.replace(cp, needs_layout_passes=False)
    if "use_tc_tiling_on_sc" in pltpu.CompilerParams.__dataclass_fields__:
        cp = dataclasses.replace(cp, use_tc_tiling_on_sc=False)

    @functools.partial(
        pl.kernel,
        compiler_params=cp,
        out_type=jax.ShapeDtypeStruct((n_tok * HID,), jnp.float32),
        mesh=mesh,
        scratch_types=[
            pltpu.VMEM((rows_w, C), jnp.int32),
            pltpu.VMEM((rows_w, C), jnp.int32),
            pltpu.VMEM((rows_w, C), jnp.int32),
            pltpu.VMEM((rows_w, C), jnp.int32),
            pltpu.VMEM((rows_w, C), jnp.int32),
            pltpu.VMEM((rows_w, C), jnp.int32),
            pltpu.VMEM((C, HID), jnp.float32),
            pltpu.VMEM((C, HID), jnp.float32),
            pltpu.VMEM((C * HID,), jnp.float32),
            pltpu.VMEM((C * HID,), jnp.float32),
            pltpu.VMEM((DEMO_VOCAB * HID,), jnp.float32),
            pltpu.VMEM((MAX_POS * HID,), jnp.float32),
            pltpu.VMEM((SEG_VOCAB * HID,), jnp.float32),
            pltpu.VMEM((HID,), jnp.float32),
            pltpu.VMEM((HID,), jnp.float32),
            pltpu.SemaphoreType.DMA,
            pltpu.SemaphoreType.DMA,
            pltpu.SemaphoreType.DMA,
            pltpu.SemaphoreType.DMA,
        ],
    )
    def k(wids_h, aids_h, bids_h, cids_h, sids_h, pids_h,
          wtab_h, dtab_h, ptab_h, stab_h, gamma_h, beta_h, out_h,
          idb_w, idb_a, idb_b, idb_c, idb_s, idb_p,
          wrows0, wrows1, obuf0, obuf1,
          dtab_v, ptab_v, stab_v, g_v, b_v,
          sem_g0, sem_g1, sem_o0, sem_o1):
        wid = lax.axis_index("s") * NC + lax.axis_index("c")
        row0 = wid * rows_w

        pltpu.sync_copy(dtab_h, dtab_v)
        pltpu.sync_copy(ptab_h, ptab_v)
        pltpu.sync_copy(stab_h, stab_v)
        pltpu.sync_copy(gamma_h, g_v)
        pltpu.sync_copy(beta_h, b_v)
        pltpu.sync_copy(wids_h.at[pl.ds(row0, rows_w)], idb_w)
        pltpu.sync_copy(aids_h.at[pl.ds(row0, rows_w)], idb_a)
        pltpu.sync_copy(bids_h.at[pl.ds(row0, rows_w)], idb_b)
        pltpu.sync_copy(cids_h.at[pl.ds(row0, rows_w)], idb_c)
        pltpu.sync_copy(sids_h.at[pl.ds(row0, rows_w)], idb_s)
        pltpu.sync_copy(pids_h.at[pl.ds(row0, rows_w)], idb_p)

        wrows = (wrows0, wrows1)
        obufs = (obuf0, obuf1)
        sem_g = (sem_g0, sem_g1)
        sem_o = (sem_o0, sem_o1)

        pltpu.async_copy(wtab_h.at[idb_w.at[0]], wrows0, sem_g0)
        pltpu.async_copy(wtab_h.at[idb_w.at[1]], wrows1, sem_g1)

        iota16 = lax.iota(jnp.int32, L16)

        def do_chunk(g, p):
            wr = wrows[p]
            ob = obufs[p]
            pltpu.make_async_copy(wtab_h.at[idb_w.at[g]], wr, sem_g[p]).wait()

            @pl.when(g >= 2)
            def _():
                pltpu.make_async_copy(
                    ob, out_h.at[pl.ds(0, C * HID)], sem_o[p]).wait()

            @pl.loop(0, C // L16)
            def _grp(gg):
                s = gg * L16
                tok16 = iota16 + s
                base_t = tok16 * HID
                aidv = idb_a[g, pl.ds(s, L16)] * HID
                bidv = idb_b[g, pl.ds(s, L16)] * HID
                cidv = idb_c[g, pl.ds(s, L16)] * HID
                sidv = idb_s[g, pl.ds(s, L16)] * HID
                pidv = idb_p[g, pl.ds(s, L16)] * HID

                s1 = jnp.zeros((L16,), jnp.float32)
                s2 = jnp.zeros((L16,), jnp.float32)
                for h in range(HID):
                    hv = jnp.full((L16,), h, dtype=jnp.int32)
                    x = (plsc.load_gather(wr, [tok16, hv])
                         + plsc.load_gather(dtab_v, [aidv + h])
                         + plsc.load_gather(dtab_v, [bidv + h])
                         + plsc.load_gather(dtab_v, [cidv + h])
                         + plsc.load_gather(ptab_v, [pidv + h])
                         + plsc.load_gather(stab_v, [sidv + h]))
                    s1 = s1 + x
                    s2 = s2 + x * x
                    plsc.store_scatter(ob, [base_t + h], x)

                mean = s1 * (1.0 / HID)
                var = s2 * (1.0 / HID) - mean * mean
                rstd = _rsqrt(var + 1e-12)

                gvec = [g_v[pl.ds(kk * L16, L16)] for kk in range(KV)]
                bvec = [b_v[pl.ds(kk * L16, L16)] for kk in range(KV)]
                for j in range(L16):
                    t = s + j
                    m = _splat(mean, j)
                    r = _splat(rstd, j)
                    for kk in range(KV):
                        o = kk * L16
                        xk = ob[pl.ds(t * HID + o, L16)]
                        ob[pl.ds(t * HID + o, L16)] = (
                            (xk - m) * (r * gvec[kk]) + bvec[kk])

            off64 = (row0 * C + g * C) * HID
            pltpu.async_copy(ob, out_h.at[pl.ds(off64, C * HID)], sem_o[p])

            @pl.when(g + 2 < nchunk)
            def _():
                pltpu.async_copy(wtab_h.at[idb_w.at[g + 2]], wr, sem_g[p])

        @pl.loop(0, nchunk // 2)
        def _pair(i):
            do_chunk(i * 2, 0)
            do_chunk(i * 2 + 1, 1)

        pltpu.make_async_copy(
            obuf0, out_h.at[pl.ds(0, C * HID)], sem_o0).wait()
        pltpu.make_async_copy(
            obuf1, out_h.at[pl.ds(0, C * HID)], sem_o1).wait()

    return k(wids2, aids2, bids2, cids2, sids2, pids2,
             wtab, dtab_f, ptab_f, stab_f, gamma, beta)


def kernel(word_ids, age_ids, bmi_ids, cycle_len_ids, seg_ids, posi_ids,
           word_table, demo_table, posi_table, seg_table, ln_gamma, ln_beta):
    b, l = word_ids.shape
    n_tok = b * l
    as_rows = lambda x: x.reshape(n_tok // C, C).astype(jnp.int32)
    out = _embed_ln(
        n_tok,
        as_rows(word_ids), as_rows(age_ids), as_rows(bmi_ids),
        as_rows(cycle_len_ids), as_rows(seg_ids), as_rows(posi_ids),
        word_table.astype(jnp.float32),
        demo_table.astype(jnp.float32).reshape(-1),
        posi_table.astype(jnp.float32).reshape(-1),
        seg_table.astype(jnp.float32).reshape(-1),
        ln_gamma.astype(jnp.float32), ln_beta.astype(jnp.float32),
    )
    return out.reshape(b, l, HID)
